# Initial kernel scaffold; baseline (speedup 1.0000x reference)
import jax
import jax.numpy as jnp
from jax.experimental import pallas as pl


def kernel(x, token_table, pos_table):
    raise NotImplementedError("write your pallas kernel here")


# speedup vs baseline: 1.2775x; 1.2775x over previous
import functools

import jax
import jax.numpy as jnp
from jax import lax
from jax.experimental import pallas as pl
from jax.experimental.pallas import tpu as pltpu
from jax.experimental.pallas import tpu_sc as plsc

MAXLEN = 2048
EMBED_DIM = 128
BATCH = 4

NUM_CORES = 2
NUM_SUBCORES = 16
NUM_WORKERS = NUM_CORES * NUM_SUBCORES
FLAT = BATCH * MAXLEN
ROWS_PER_WORKER = FLAT // NUM_WORKERS
CHUNK = 128
CHUNKS_PER_WORKER = ROWS_PER_WORKER // CHUNK
POS_SPANS = MAXLEN // ROWS_PER_WORKER


def _emb_body(x_hbm, table_hbm, pos_hbm, out_hbm, idx_v, rows_v, pos_v, sem):
    c = lax.axis_index("c")
    s = lax.axis_index("s")
    w = s * NUM_CORES + c

    pltpu.sync_copy(x_hbm.at[pl.ds(w * CHUNKS_PER_WORKER, CHUNKS_PER_WORKER)],
                    idx_v)

    cps = []
    for j in range(CHUNKS_PER_WORKER):
        cps.append(pltpu.async_copy(
            table_hbm.at[idx_v.at[j]],
            rows_v.at[pl.ds(j * CHUNK, CHUNK)],
            sem))

    pos_base = (w % POS_SPANS) * ROWS_PER_WORKER
    pltpu.sync_copy(pos_hbm.at[pl.ds(pos_base, ROWS_PER_WORKER)], pos_v)

    for cp in cps:
        cp.wait()

    def add_row(i, carry):
        for j in range(EMBED_DIM // 16):
            sl = (i, pl.ds(j * 16, 16))
            rows_v[sl] = rows_v[sl] + pos_v[sl]
        return carry

    lax.fori_loop(0, ROWS_PER_WORKER, add_row, 0)

    pltpu.sync_copy(rows_v, out_hbm.at[pl.ds(w * ROWS_PER_WORKER,
                                             ROWS_PER_WORKER)])


@jax.jit
def _embed(x2, token_table, pos_table):
    mesh = plsc.VectorSubcoreMesh(core_axis_name="c", subcore_axis_name="s")
    run = functools.partial(
        pl.kernel,
        mesh=mesh,
        out_type=jax.ShapeDtypeStruct((FLAT, EMBED_DIM), jnp.float32),
        scratch_types=[
            pltpu.VMEM((CHUNKS_PER_WORKER, CHUNK), jnp.int32),
            pltpu.VMEM((ROWS_PER_WORKER, EMBED_DIM), jnp.float32),
            pltpu.VMEM((ROWS_PER_WORKER, EMBED_DIM), jnp.float32),
            pltpu.SemaphoreType.DMA,
        ],
    )(_emb_body)
    return run(x2, token_table, pos_table)


def kernel(x, token_table, pos_table):
    x2 = x.astype(jnp.int32).reshape(FLAT // CHUNK, CHUNK)
    out = _embed(x2, token_table, pos_table)
    return out.reshape(BATCH, MAXLEN, EMBED_DIM)


# speedup vs baseline: 1.2929x; 1.0121x over previous
import functools

import jax
import jax.numpy as jnp
from jax import lax
from jax.experimental import pallas as pl
from jax.experimental.pallas import tpu as pltpu
from jax.experimental.pallas import tpu_sc as plsc

MAXLEN = 2048
EMBED_DIM = 128
BATCH = 4

NUM_CORES = 2
NUM_SUBCORES = 16
NUM_WORKERS = NUM_CORES * NUM_SUBCORES
FLAT = BATCH * MAXLEN
ROWS_PER_WORKER = FLAT // NUM_WORKERS
CHUNK = 128
CHUNKS_PER_WORKER = ROWS_PER_WORKER // CHUNK
POS_SPANS = MAXLEN // ROWS_PER_WORKER


ROW_UNROLL = 2


def _emb_body(x_hbm, table_hbm, pos_hbm, out_hbm, idx_v, rows_v, pos_v,
              sem_g0, sem_g1, sem_out):
    c = lax.axis_index("c")
    s = lax.axis_index("s")
    w = s * NUM_CORES + c

    pltpu.sync_copy(x_hbm.at[pl.ds(w * CHUNKS_PER_WORKER, CHUNKS_PER_WORKER)],
                    idx_v)

    sems = (sem_g0, sem_g1)
    cps = []
    for j in range(CHUNKS_PER_WORKER):
        cps.append(pltpu.async_copy(
            table_hbm.at[idx_v.at[j]],
            rows_v.at[pl.ds(j * CHUNK, CHUNK)],
            sems[j]))

    pos_base = (w % POS_SPANS) * ROWS_PER_WORKER
    pltpu.sync_copy(pos_hbm.at[pl.ds(pos_base, ROWS_PER_WORKER)], pos_v)

    out_cps = []
    for j in range(CHUNKS_PER_WORKER):
        cps[j].wait()
        base = j * CHUNK

        def add_rows(i, carry, base=base):
            for u in range(ROW_UNROLL):
                r = base + i * ROW_UNROLL + u
                for k in range(EMBED_DIM // 16):
                    sl = (r, pl.ds(k * 16, 16))
                    plsc.addupdate(rows_v.at[sl], pos_v[sl])
            return carry

        lax.fori_loop(0, CHUNK // ROW_UNROLL, add_rows, 0)
        out_cps.append(pltpu.async_copy(
            rows_v.at[pl.ds(base, CHUNK)],
            out_hbm.at[pl.ds(w * ROWS_PER_WORKER + base, CHUNK)],
            sem_out))

    for cp in out_cps:
        cp.wait()


@jax.jit
def _embed(x2, token_table, pos_table):
    mesh = plsc.VectorSubcoreMesh(core_axis_name="c", subcore_axis_name="s")
    run = functools.partial(
        pl.kernel,
        mesh=mesh,
        out_type=jax.ShapeDtypeStruct((FLAT, EMBED_DIM), jnp.float32),
        scratch_types=[
            pltpu.VMEM((CHUNKS_PER_WORKER, CHUNK), jnp.int32),
            pltpu.VMEM((ROWS_PER_WORKER, EMBED_DIM), jnp.float32),
            pltpu.VMEM((ROWS_PER_WORKER, EMBED_DIM), jnp.float32),
            pltpu.SemaphoreType.DMA,
            pltpu.SemaphoreType.DMA,
            pltpu.SemaphoreType.DMA,
        ],
    )(_emb_body)
    return run(x2, token_table, pos_table)


def kernel(x, token_table, pos_table):
    x2 = x.astype(jnp.int32).reshape(FLAT // CHUNK, CHUNK)
    out = _embed(x2, token_table, pos_table)
    return out.reshape(BATCH, MAXLEN, EMBED_DIM)
